# Initial kernel scaffold; baseline (speedup 1.0000x reference)
import jax
import jax.numpy as jnp
from jax.experimental import pallas as pl


def kernel(x, edge_index, W_node_to_edge, W_edge):
    raise NotImplementedError("write your pallas kernel here")


# speedup vs baseline: 2.7718x; 2.7718x over previous
import functools

import jax
import jax.numpy as jnp
from jax import lax
from jax.experimental import pallas as pl
from jax.experimental.pallas import tpu as pltpu
from jax.experimental.pallas import tpu_sc as plsc


def _dense_body(x_ref, w1_ref, w2_ref, y_ref):
    t = lax.dot_general(
        x_ref[...], w1_ref[...], (((1,), (1,)), ((), ())),
        preferred_element_type=jnp.float32)
    y_ref[...] = lax.dot_general(
        t, w2_ref[...], (((1,), (1,)), ((), ())),
        preferred_element_type=jnp.float32)


def _node_transform(x, w1, w2):
    n, _ = x.shape
    out_ch = w2.shape[0]
    return pl.pallas_call(
        _dense_body,
        out_shape=jax.ShapeDtypeStruct((n, out_ch), jnp.float32),
    )(x, w1, w2)


def _make_sc_gather(n_edges, d, chunk):
    info = plsc.get_sparse_core_info()
    nc, ns = info.num_cores, info.num_subcores
    nw = nc * ns
    assert n_edges % nw == 0
    per_w = n_edges // nw
    assert per_w % chunk == 0 and chunk % 8 == 0 and chunk <= 128
    n_chunks = per_w // chunk
    mesh = plsc.VectorSubcoreMesh(core_axis_name="c", subcore_axis_name="s")

    @functools.partial(
        pl.kernel,
        out_type=jax.ShapeDtypeStruct((n_edges, d), jnp.float32),
        mesh=mesh,
        scratch_types=[
            pltpu.VMEM((chunk,), jnp.int32),
            pltpu.VMEM((chunk, d), jnp.float32),
            pltpu.SemaphoreType.DMA,
        ],
    )
    def gather(y_hbm, row_hbm, out_hbm, idx_v, rows_v, sem):
        wid = lax.axis_index("s") * nc + lax.axis_index("c")
        base = wid * per_w

        def step(i, carry):
            off = base + i * chunk
            pltpu.sync_copy(row_hbm.at[pl.ds(off, chunk)], idx_v)
            pltpu.async_copy(y_hbm.at[idx_v], rows_v, sem).wait()
            pltpu.sync_copy(rows_v, out_hbm.at[pl.ds(off, chunk)])
            return carry

        lax.fori_loop(0, n_chunks, step, 0)

    return gather


def kernel(x, edge_index, W_node_to_edge, W_edge):
    row = edge_index[0].astype(jnp.int32)
    y = _node_transform(x, W_node_to_edge, W_edge)
    n_edges = row.shape[0]
    d = y.shape[1]
    gather = _make_sc_gather(n_edges, d, chunk=80)
    return gather(y, row)


# speedup vs baseline: 5.4440x; 1.9640x over previous
import functools

import jax
import jax.numpy as jnp
from jax import lax
from jax.experimental import pallas as pl
from jax.experimental.pallas import tpu as pltpu
from jax.experimental.pallas import tpu_sc as plsc


def _dense_body(x_ref, w1_ref, w2_ref, y_ref):
    t = lax.dot_general(
        x_ref[...], w1_ref[...], (((1,), (1,)), ((), ())),
        preferred_element_type=jnp.float32)
    y_ref[...] = lax.dot_general(
        t, w2_ref[...], (((1,), (1,)), ((), ())),
        preferred_element_type=jnp.float32)


def _node_transform(x, w1, w2):
    n, _ = x.shape
    out_ch = w2.shape[0]
    return pl.pallas_call(
        _dense_body,
        out_shape=jax.ShapeDtypeStruct((n, out_ch), jnp.float32),
    )(x, w1, w2)


def _make_sc_gather(n_edges, d, chunk, nbuf, look):
    info = plsc.get_sparse_core_info()
    nc, ns = info.num_cores, info.num_subcores
    nw = nc * ns
    assert n_edges % nw == 0
    per_w = n_edges // nw
    assert per_w % chunk == 0 and chunk % 8 == 0 and chunk <= 128
    n_chunks = per_w // chunk
    assert n_chunks % nbuf == 0 and n_chunks >= nbuf and 0 < look < nbuf
    mesh = plsc.VectorSubcoreMesh(core_axis_name="c", subcore_axis_name="s")

    @functools.partial(
        pl.kernel,
        out_type=jax.ShapeDtypeStruct((n_edges, d), jnp.float32),
        mesh=mesh,
        scratch_types=(
            [pltpu.VMEM((per_w,), jnp.int32),
             pltpu.VMEM((nbuf, chunk, d), jnp.float32)]
            + [pltpu.SemaphoreType.DMA] * (2 * nbuf)
        ),
    )
    def gather(y_hbm, row_hbm, out_hbm, idx_v, rows_v, *sems):
        gsem, ssem = sems[:nbuf], sems[nbuf:]
        wid = lax.axis_index("s") * nc + lax.axis_index("c")
        base = wid * per_w
        pltpu.sync_copy(row_hbm.at[pl.ds(base, per_w)], idx_v)

        def issue_gather(j, b):
            pltpu.async_copy(
                y_hbm.at[idx_v.at[pl.ds(j * chunk, chunk)]],
                rows_v.at[b], gsem[b])

        def wait_gather(b):
            pltpu.make_async_copy(
                y_hbm.at[idx_v.at[pl.ds(0, chunk)]],
                rows_v.at[b], gsem[b]).wait()

        def issue_store(i, b):
            pltpu.async_copy(
                rows_v.at[b], out_hbm.at[pl.ds(base + i * chunk, chunk)],
                ssem[b])

        def wait_store(b):
            pltpu.make_async_copy(
                rows_v.at[b], out_hbm.at[pl.ds(base, chunk)],
                ssem[b]).wait()

        for b in range(look):
            issue_gather(b, b)

        @pl.loop(0, n_chunks, step=nbuf)
        def _(g):
            for b in range(nbuf):
                i = g + b
                bj = (b + look) % nbuf

                @pl.when(i + look < n_chunks)
                def _():
                    @pl.when(i >= nbuf - look)
                    def _():
                        wait_store(bj)
                    issue_gather(i + look, bj)

                wait_gather(b)
                issue_store(i, b)

        for b in range(nbuf):
            wait_store(b)

    return gather


def kernel(x, edge_index, W_node_to_edge, W_edge):
    row = edge_index[0].astype(jnp.int32)
    y = _node_transform(x, W_node_to_edge, W_edge)
    n_edges = row.shape[0]
    d = y.shape[1]
    gather = _make_sc_gather(n_edges, d, chunk=80, nbuf=5, look=3)
    return gather(y, row)


# speedup vs baseline: 5.4588x; 1.0027x over previous
import functools

import jax
import jax.numpy as jnp
from jax import lax
from jax.experimental import pallas as pl
from jax.experimental.pallas import tpu as pltpu
from jax.experimental.pallas import tpu_sc as plsc


def _dense_body(x_ref, w1_ref, w2_ref, y_ref):
    t = lax.dot_general(
        x_ref[...], w1_ref[...], (((1,), (1,)), ((), ())),
        preferred_element_type=jnp.float32)
    y_ref[...] = lax.dot_general(
        t, w2_ref[...], (((1,), (1,)), ((), ())),
        preferred_element_type=jnp.float32)


def _node_transform(x, w1, w2):
    n, _ = x.shape
    out_ch = w2.shape[0]
    return pl.pallas_call(
        _dense_body,
        out_shape=jax.ShapeDtypeStruct((n, out_ch), jnp.float32),
    )(x, w1, w2)


def _make_sc_gather(n_edges, d, chunk, nbuf, look):
    info = plsc.get_sparse_core_info()
    nc, ns = info.num_cores, info.num_subcores
    nw = nc * ns
    assert n_edges % nw == 0
    per_w = n_edges // nw
    assert per_w % chunk == 0 and chunk % 8 == 0 and chunk <= 128
    n_chunks = per_w // chunk
    assert n_chunks % nbuf == 0 and n_chunks >= nbuf and 0 < look < nbuf
    mesh = plsc.VectorSubcoreMesh(core_axis_name="c", subcore_axis_name="s")

    @functools.partial(
        pl.kernel,
        out_type=jax.ShapeDtypeStruct((n_edges, d), jnp.float32),
        mesh=mesh,
        scratch_types=(
            [pltpu.VMEM((per_w,), jnp.int32),
             pltpu.VMEM((nbuf, chunk, d), jnp.float32)]
            + [pltpu.SemaphoreType.DMA] * (2 * nbuf)
        ),
    )
    def gather(y_hbm, row_hbm, out_hbm, idx_v, rows_v, *sems):
        gsem, ssem = sems[:nbuf], sems[nbuf:]
        wid = lax.axis_index("s") * nc + lax.axis_index("c")
        base = wid * per_w
        pltpu.sync_copy(row_hbm.at[pl.ds(base, per_w)], idx_v)

        def issue_gather(j, b):
            pltpu.async_copy(
                y_hbm.at[idx_v.at[pl.ds(j * chunk, chunk)]],
                rows_v.at[b], gsem[b])

        def wait_gather(b):
            pltpu.make_async_copy(
                y_hbm.at[idx_v.at[pl.ds(0, chunk)]],
                rows_v.at[b], gsem[b]).wait()

        def issue_store(i, b):
            pltpu.async_copy(
                rows_v.at[b], out_hbm.at[pl.ds(base + i * chunk, chunk)],
                ssem[b])

        def wait_store(b):
            pltpu.make_async_copy(
                rows_v.at[b], out_hbm.at[pl.ds(base, chunk)],
                ssem[b]).wait()

        for b in range(look):
            issue_gather(b, b)

        @pl.loop(0, n_chunks, step=nbuf)
        def _(g):
            for b in range(nbuf):
                i = g + b
                bj = (b + look) % nbuf

                @pl.when(i + look < n_chunks)
                def _():
                    @pl.when(i >= nbuf - look)
                    def _():
                        wait_store(bj)
                    issue_gather(i + look, bj)

                wait_gather(b)
                issue_store(i, b)

        for b in range(nbuf):
            wait_store(b)

    return gather


def kernel(x, edge_index, W_node_to_edge, W_edge):
    row = edge_index[0].astype(jnp.int32)
    y = _node_transform(x, W_node_to_edge, W_edge)
    n_edges = row.shape[0]
    d = y.shape[1]
    gather = _make_sc_gather(n_edges, d, chunk=40, nbuf=10, look=5)
    return gather(y, row)


# speedup vs baseline: 8.0315x; 1.4713x over previous
import functools

import jax
import jax.numpy as jnp
from jax import lax
from jax.experimental import pallas as pl
from jax.experimental.pallas import tpu as pltpu
from jax.experimental.pallas import tpu_sc as plsc


def _dense_body(x_ref, w1_ref, w2_ref, y_ref):
    t = lax.dot_general(
        x_ref[...], w1_ref[...], (((1,), (1,)), ((), ())),
        preferred_element_type=jnp.float32)
    y_ref[...] = lax.dot_general(
        t, w2_ref[...], (((1,), (1,)), ((), ())),
        preferred_element_type=jnp.float32)


def _node_transform(x, w1, w2):
    n, _ = x.shape
    out_ch = w2.shape[0]
    return pl.pallas_call(
        _dense_body,
        out_shape=jax.ShapeDtypeStruct((n, out_ch), jnp.float32),
    )(x, w1, w2)


def _make_sc_gather(n_nodes, n_edges, d, chunk, nbuf, look):
    info = plsc.get_sparse_core_info()
    nc, ns = info.num_cores, info.num_subcores
    nw = nc * ns
    assert n_edges % nw == 0
    per_w = n_edges // nw
    assert per_w % chunk == 0 and chunk % 8 == 0 and chunk <= 128
    n_chunks = per_w // chunk
    assert n_chunks % nbuf == 0 and n_chunks >= nbuf and 0 < look < nbuf
    n_stagers = ns
    while n_nodes % n_stagers or (n_nodes // n_stagers) % 8:
        n_stagers -= 1
    stage_rows = n_nodes // n_stagers
    mesh = plsc.VectorSubcoreMesh(core_axis_name="c", subcore_axis_name="s")

    @functools.partial(
        pl.kernel,
        out_type=jax.ShapeDtypeStruct((n_edges, d), jnp.float32),
        mesh=mesh,
        scratch_types=(
            [pltpu.VMEM((per_w,), jnp.int32),
             pltpu.VMEM((nbuf, chunk, d), jnp.float32),
             pltpu.VMEM_SHARED((n_nodes, d), jnp.float32)]
            + [pltpu.SemaphoreType.DMA] * (2 * nbuf)
        ),
    )
    def gather(y_hbm, row_hbm, out_hbm, idx_v, rows_v, y_sp, *sems):
        gsem, ssem = sems[:nbuf], sems[nbuf:]
        sid = lax.axis_index("s")
        wid = sid * nc + lax.axis_index("c")
        base = wid * per_w
        @pl.when(sid < n_stagers)
        def _():
            off = pl.multiple_of(sid * stage_rows, 8)
            pltpu.sync_copy(y_hbm.at[pl.ds(off, stage_rows)],
                            y_sp.at[pl.ds(off, stage_rows)])

        pltpu.sync_copy(row_hbm.at[pl.ds(base, per_w)], idx_v)
        plsc.subcore_barrier()

        def issue_gather(j, b):
            pltpu.async_copy(
                y_sp.at[idx_v.at[pl.ds(j * chunk, chunk)]],
                rows_v.at[b], gsem[b])

        def wait_gather(b):
            pltpu.make_async_copy(
                y_sp.at[idx_v.at[pl.ds(0, chunk)]],
                rows_v.at[b], gsem[b]).wait()

        def issue_store(i, b):
            pltpu.async_copy(
                rows_v.at[b], out_hbm.at[pl.ds(base + i * chunk, chunk)],
                ssem[b])

        def wait_store(b):
            pltpu.make_async_copy(
                rows_v.at[b], out_hbm.at[pl.ds(base, chunk)],
                ssem[b]).wait()

        for b in range(look):
            issue_gather(b, b)

        @pl.loop(0, n_chunks, step=nbuf)
        def _(g):
            for b in range(nbuf):
                i = g + b
                bj = (b + look) % nbuf

                @pl.when(i + look < n_chunks)
                def _():
                    @pl.when(i >= nbuf - look)
                    def _():
                        wait_store(bj)
                    issue_gather(i + look, bj)

                wait_gather(b)
                issue_store(i, b)

        for b in range(nbuf):
            wait_store(b)

    return gather


def kernel(x, edge_index, W_node_to_edge, W_edge):
    row = edge_index[0].astype(jnp.int32)
    y = _node_transform(x, W_node_to_edge, W_edge)
    n_edges = row.shape[0]
    d = y.shape[1]
    gather = _make_sc_gather(y.shape[0], n_edges, d, chunk=40, nbuf=5, look=3)
    return gather(y, row)


# speedup vs baseline: 8.0967x; 1.0081x over previous
import functools

import jax
import jax.numpy as jnp
from jax import lax
from jax.experimental import pallas as pl
from jax.experimental.pallas import tpu as pltpu
from jax.experimental.pallas import tpu_sc as plsc


def _dense_body(x_ref, w1_ref, w2_ref, y_ref):
    wc = lax.dot_general(
        w2_ref[...], w1_ref[...], (((1,), (0,)), ((), ())),
        preferred_element_type=jnp.float32)
    y_ref[...] = lax.dot_general(
        x_ref[...], wc, (((1,), (1,)), ((), ())),
        preferred_element_type=jnp.float32)


def _node_transform(x, w1, w2):
    n, _ = x.shape
    out_ch = w2.shape[0]
    return pl.pallas_call(
        _dense_body,
        out_shape=jax.ShapeDtypeStruct((n, out_ch), jnp.float32),
    )(x, w1, w2)


def _make_sc_gather(n_nodes, n_edges, d, chunk, nbuf, look):
    info = plsc.get_sparse_core_info()
    nc, ns = info.num_cores, info.num_subcores
    nw = nc * ns
    assert n_edges % nw == 0
    per_w = n_edges // nw
    assert per_w % chunk == 0 and chunk % 8 == 0 and chunk <= 128
    n_chunks = per_w // chunk
    assert n_chunks % nbuf == 0 and n_chunks >= nbuf and 0 < look < nbuf
    n_stagers = ns
    while n_nodes % n_stagers or (n_nodes // n_stagers) % 8:
        n_stagers -= 1
    stage_rows = n_nodes // n_stagers
    mesh = plsc.VectorSubcoreMesh(core_axis_name="c", subcore_axis_name="s")

    @functools.partial(
        pl.kernel,
        out_type=jax.ShapeDtypeStruct((n_edges, d), jnp.float32),
        mesh=mesh,
        scratch_types=(
            [pltpu.VMEM((per_w,), jnp.int32),
             pltpu.VMEM((nbuf, chunk, d), jnp.float32),
             pltpu.VMEM_SHARED((n_nodes, d), jnp.float32)]
            + [pltpu.SemaphoreType.DMA] * (2 * nbuf)
        ),
    )
    def gather(y_hbm, row_hbm, out_hbm, idx_v, rows_v, y_sp, *sems):
        gsem, ssem = sems[:nbuf], sems[nbuf:]
        sid = lax.axis_index("s")
        wid = sid * nc + lax.axis_index("c")
        base = wid * per_w
        @pl.when(sid < n_stagers)
        def _():
            off = pl.multiple_of(sid * stage_rows, 8)
            pltpu.sync_copy(y_hbm.at[pl.ds(off, stage_rows)],
                            y_sp.at[pl.ds(off, stage_rows)])

        pltpu.sync_copy(row_hbm.at[pl.ds(base, per_w)], idx_v)
        plsc.subcore_barrier()

        def issue_gather(j, b):
            pltpu.async_copy(
                y_sp.at[idx_v.at[pl.ds(j * chunk, chunk)]],
                rows_v.at[b], gsem[b])

        def wait_gather(b):
            pltpu.make_async_copy(
                y_sp.at[idx_v.at[pl.ds(0, chunk)]],
                rows_v.at[b], gsem[b]).wait()

        def issue_store(i, b):
            pltpu.async_copy(
                rows_v.at[b], out_hbm.at[pl.ds(base + i * chunk, chunk)],
                ssem[b])

        def wait_store(b):
            pltpu.make_async_copy(
                rows_v.at[b], out_hbm.at[pl.ds(base, chunk)],
                ssem[b]).wait()

        for b in range(look):
            issue_gather(b, b)

        @pl.loop(0, n_chunks, step=nbuf)
        def _(g):
            for b in range(nbuf):
                i = g + b
                bj = (b + look) % nbuf

                @pl.when(i + look < n_chunks)
                def _():
                    @pl.when(i >= nbuf - look)
                    def _():
                        wait_store(bj)
                    issue_gather(i + look, bj)

                wait_gather(b)
                issue_store(i, b)

        for b in range(nbuf):
            wait_store(b)

    return gather


def kernel(x, edge_index, W_node_to_edge, W_edge):
    row = edge_index[0].astype(jnp.int32)
    y = _node_transform(x, W_node_to_edge, W_edge)
    n_edges = row.shape[0]
    d = y.shape[1]
    gather = _make_sc_gather(y.shape[0], n_edges, d, chunk=40, nbuf=5, look=3)
    return gather(y, row)
